# baseline (speedup 1.0000x reference)
import functools

import jax
import jax.numpy as jnp
from jax import lax
from jax.experimental import pallas as pl
from jax.experimental.pallas import tpu as pltpu
from jax.experimental.pallas import tpu_sc as plsc

E = 64
L = 16
NC = 2
NS = 16
NW = NC * NS
MARGIN = 1.0
ALPHA = 0.2
EPS = 1e-7

SW = 384
E_WIN = 81
U_WIN = 8
E_TW = E_WIN * SW
U_TW = U_WIN * SW
E_N = 1000000
U_N = 100000
E_CAP = 640
U_CAP = 1024
E_MCAP = 32
U_MCAP = 128
SPARE = 16384
CH = 1024


def _gather_body(u, i, hp, tp, hn, tn, ut, et, ut_tail, et_tail,
                 rows_u, rows_i, rows_hp, rows_tp, rows_hn, rows_tn,
                 istg6,
                 ecol_i, eslot_i, ecol_hp, eslot_hp, ecol_tp, eslot_tp,
                 ecol_hn, eslot_hn, ecol_tn, eslot_tn,
                 ucol, uslot,
                 wbuf0, wbuf1, twbuf_e, twbuf_u,
                 st_i, st_hp, st_tp, st_hn, st_tn, st_u,
                 mcol, mslot, sl_i, sl_hp, sl_tp, sl_hn, sl_tn, slotu,
                 isem, wsem, ssem):
    batch = u.shape[0]
    wid = lax.axis_index("s") * NC + lax.axis_index("c")
    iota = lax.iota(jnp.int32, L)
    zero_v = jnp.zeros((L,), jnp.int32)
    is_last = wid == NW - 1

    e_lo = wid * E_TW
    u_lo = wid * U_TW

    esets = ((i, ecol_i, eslot_i, st_i, rows_i, sl_i),
             (hp, ecol_hp, eslot_hp, st_hp, rows_hp, sl_hp),
             (tp, ecol_tp, eslot_tp, st_tp, rows_tp, sl_tp),
             (hn, ecol_hn, eslot_hn, st_hn, rows_hn, sl_hn),
             (tn, ecol_tn, eslot_tn, st_tn, rows_tn, sl_tn))
    uset = (u, ucol, uslot, st_u, rows_u, slotu)
    all6 = esets + (uset,)

    n_ch = batch // CH
    e_lo_v = zero_v + e_lo
    u_lo_v = zero_v + u_lo
    e_hi_v = jnp.where(is_last, zero_v + E_N, e_lo_v + E_TW)
    u_hi_v = jnp.where(is_last, zero_v + U_N, u_lo_v + U_TW)
    bounds = [(e_lo_v, e_hi_v)] * 5 + [(u_lo_v, u_hi_v)]

    def ch_body(ch, offs):
        c0 = pl.multiple_of(ch * CH, CH)
        for k6, (arr, _, _, _, _, _) in enumerate(all6):
            pltpu.sync_copy(arr.at[pl.ds(c0, CH)], istg6.at[k6], )

        def body(v, offs):
            new = []
            for k6, (_, pcol, pslot, _, _, _) in enumerate(all6):
                lo_v, hi_v = bounds[k6]
                vals = istg6[k6, pl.ds(v * L, L)]
                mask = (vals >= lo_v) & (vals < hi_v)
                mi = jnp.where(mask, 1, 0).astype(jnp.int32)
                cs = plsc.cumsum(mi)
                pos = offs[k6] + cs - 1
                plsc.store_scatter(pcol, [pos], vals - lo_v, mask=mask)
                slot = (zero_v + ch * CH) + v * L + iota
                plsc.store_scatter(pslot, [pos], slot, mask=mask)
                new.append(offs[k6] + plsc.all_reduce_population_count(mask))
            return tuple(new)

        return lax.fori_loop(0, CH // L, body, offs)

    offs = lax.fori_loop(0, n_ch, ch_body, (zero_v,) * 6)
    e_counts = [jnp.max(o) for o in offs[:5]]
    n_u = jnp.max(offs[5])

    def fine_scan(sets, cnts, filt_lo, filt_w, buf_lo, cap_vecs):
        lo_v = zero_v + filt_lo
        hi_v = lo_v + filt_w
        blo_v = zero_v + buf_lo
        nsets = len(sets)

        def body(v, offs):
            new = []
            for j, ((_, pcol, pslot, _, _, _), n_k) in enumerate(
                    zip(sets, cnts)):
                vals_ok = (v * L + iota) < (zero_v + n_k)
                cols = pcol[pl.ds(v * L, L)]
                slots = pslot[pl.ds(v * L, L)]
                mask = (cols >= lo_v) & (cols < hi_v) & vals_ok
                mi = jnp.where(mask, 1, 0).astype(jnp.int32)
                cs = plsc.cumsum(mi)
                pos = (offs[j] + cs - 1) + j * E_MCAP if nsets > 1 else (
                    offs[j] + cs - 1)
                plsc.store_scatter(mcol, [pos], cols - blo_v, mask=mask)
                plsc.store_scatter(mslot, [pos], slots, mask=mask)
                new.append(offs[j] + plsc.all_reduce_population_count(mask))
            return tuple(new)

        offs = lax.fori_loop(0, cap_vecs, body, (zero_v,) * nsets)
        return [jnp.max(o) for o in offs]

    def gather_and_stage(wbuf, sets, ms, mcap, mbase_stride):
        for j, ((_, _, _, stg, _, slotbuf), m) in enumerate(zip(sets, ms)):
            m_v = zero_v + m
            mb = j * mbase_stride

            def gather_group(g, stg=stg, m_v=m_v, mb=mb):
                lane = (zero_v + g * L) + iota
                sel = lane < m_v
                colv = jnp.where(sel, mcol[pl.ds(mb + g * L, L)], 0)

                def d_body(dd, _):
                    for k in range(4):
                        dcol = zero_v + (dd * 4 + k)
                        val = plsc.load_gather(wbuf, [dcol, colv])
                        plsc.store_scatter(stg, [lane, dcol], val)
                    return 0

                lax.fori_loop(0, E // 4, d_body, 0)

            gather_group(0)
            for g in range(1, mcap // L):
                @pl.when(m > g * L)
                def _(g=g):
                    gather_group(g)

            def slot_body(g, _, slotbuf=slotbuf, m_v=m_v, mb=mb):
                lane = (zero_v + g * L) + iota
                sel = lane < m_v
                goff = pl.multiple_of(g * L, L)
                slotv = jnp.where(
                    sel, mslot[pl.ds(mb + goff, L)], zero_v + SPARE)
                slotbuf[pl.ds(goff, L)] = slotv
                return 0

            lax.fori_loop(0, mcap // L, slot_body, 0)

    def issue_scatters(sets):
        return [pltpu.async_copy(stg, rows_out.at[slotbuf], ssem)
                for (_, _, _, stg, rows_out, slotbuf) in sets]

    def drain_scatters(sets):
        for (_, _, _, stg, rows_out, slotbuf) in sets:
            pltpu.make_async_copy(stg, rows_out.at[slotbuf], ssem).wait()

    e_nwin = jnp.where(is_last, E_WIN + 12, E_WIN)
    ecap_vecs = E_CAP // L

    def e_slice(w):
        wloc = pl.multiple_of(w * SW, SW)
        return wloc, et.at[:, pl.ds(e_lo + wloc, SW)]

    _, src0 = e_slice(0)
    pltpu.async_copy(src0, wbuf0, wsem)

    def e_process(wbuf, wloc):
        ms = fine_scan(esets, e_counts, wloc, SW, wloc, ecap_vecs)
        gather_and_stage(wbuf, esets, ms, E_MCAP, E_MCAP)
        issue_scatters(esets)

    def e_win_body(w, _):
        wloc, src = e_slice(w)
        buf = jnp.equal(lax.rem(w, 2), 0)

        @pl.when(w + 1 < e_nwin)
        def _():
            _, nsrc = e_slice(w + 1)

            @pl.when(buf)
            def _():
                pltpu.async_copy(nsrc, wbuf1, wsem)

            @pl.when(jnp.logical_not(buf))
            def _():
                pltpu.async_copy(nsrc, wbuf0, wsem)

        @pl.when(w > 0)
        def _():
            drain_scatters(esets)

        @pl.when(buf)
        def _():
            pltpu.make_async_copy(src, wbuf0, wsem).wait()
            e_process(wbuf0, wloc)

        @pl.when(jnp.logical_not(buf))
        def _():
            pltpu.make_async_copy(src, wbuf1, wsem).wait()
            e_process(wbuf1, wloc)

        return 0

    lax.fori_loop(0, e_nwin, e_win_body, 0)
    drain_scatters(esets)

    u_nwin = jnp.where(is_last, U_WIN + 4, U_WIN)
    ucap_vecs = U_CAP // L

    def u_win_body(w, _):
        wloc = pl.multiple_of(w * SW, SW)
        pltpu.async_copy(ut.at[:, pl.ds(u_lo + wloc, SW)], wbuf1, wsem).wait()
        ms = fine_scan((uset,), [n_u], wloc, SW, wloc, ucap_vecs)
        gather_and_stage(wbuf1, (uset,), ms, U_MCAP, U_MCAP)
        for cp in issue_scatters((uset,)):
            cp.wait()
        return 0

    lax.fori_loop(0, u_nwin, u_win_body, 0)

    @pl.when(is_last)
    def _():
        ebase = (NW - 1) * E_TW
        pltpu.sync_copy(et_tail, twbuf_e)
        ms = fine_scan(esets, e_counts, 999936 - ebase, E_N - 999936,
                       999936 - ebase, ecap_vecs)
        gather_and_stage(twbuf_e, esets, ms, E_MCAP, E_MCAP)
        for cp in issue_scatters(esets):
            cp.wait()

        ubase = (NW - 1) * U_TW
        pltpu.async_copy(ut.at[:, pl.ds(99840, 128)],
                         wbuf0.at[:, pl.ds(0, 128)], wsem).wait()
        ms = fine_scan((uset,), [n_u], 99840 - ubase, 128, 99840 - ubase,
                       ucap_vecs)
        gather_and_stage(wbuf0, (uset,), ms, U_MCAP, U_MCAP)
        for cp in issue_scatters((uset,)):
            cp.wait()

        pltpu.sync_copy(ut_tail, twbuf_u)
        ms = fine_scan((uset,), [n_u], 99968 - ubase, U_N - 99968,
                       99968 - ubase, ucap_vecs)
        gather_and_stage(twbuf_u, (uset,), ms, U_MCAP, U_MCAP)
        for cp in issue_scatters((uset,)):
            cp.wait()


def _make_gather_call(batch):
    mesh = plsc.VectorSubcoreMesh(core_axis_name="c", subcore_axis_name="s")
    f32 = jnp.float32
    i32 = jnp.int32
    rows_t = jax.ShapeDtypeStruct((batch + 128, 2 * E), f32)
    return pl.kernel(
        _gather_body,
        out_type=[rows_t] * 6,
        mesh=mesh,
        compiler_params=pltpu.CompilerParams(
            needs_layout_passes=False, use_tc_tiling_on_sc=True),
        scratch_types=(
            [pltpu.VMEM((6, CH), i32)]
            + [pltpu.VMEM((E_CAP,), i32)] * 10
            + [pltpu.VMEM((U_CAP,), i32)] * 2
            + [pltpu.VMEM((E, SW), f32)] * 2
            + [pltpu.VMEM((E, E_N - 999936), f32)]
            + [pltpu.VMEM((E, U_N - 99968), f32)]
            + [pltpu.VMEM((E_MCAP, 2 * E), f32)] * 5
            + [pltpu.VMEM((U_MCAP, 2 * E), f32)]
            + [pltpu.VMEM((5 * E_MCAP,), i32)] * 2
            + [pltpu.VMEM((E_MCAP,), i32)] * 5
            + [pltpu.VMEM((U_MCAP,), i32)]
            + [pltpu.SemaphoreType.DMA] * 3
        ),
    )


CBR = 8


def _tc_body(rp_ref, rn_ref, y_ref, u_ref, i_ref, hp_ref, tp_ref, hn_ref,
             tn_ref, re_ref, o_ref):
    k = pl.program_id(0)
    nsteps = pl.num_programs(0)

    u3 = u_ref[:, :, :E]
    i3 = i_ref[:, :, :E]
    hp3 = hp_ref[:, :, :E]
    tp3 = tp_ref[:, :, :E]
    hn3 = hn_ref[:, :, :E]
    tn3 = tn_ref[:, :, :E]
    r_tab = re_ref[...]

    rp = rp_ref[...]
    rn = rn_ref[...]
    rcols = lax.broadcasted_iota(jnp.int32, (CBR, 128, E), 2)
    oh_p = (rp[:, :, None] == rcols).astype(jnp.float32)
    oh_n = (rn[:, :, None] == rcols).astype(jnp.float32)
    dn_num = (((2,), (0,)), ((), ()))
    re_p = lax.dot_general(oh_p, r_tab, dn_num,
                           preferred_element_type=jnp.float32)
    re_n = lax.dot_general(oh_n, r_tab, dn_num,
                           preferred_element_type=jnp.float32)

    s = jnp.sum(u3 * i3, axis=2)
    yp = jnp.clip(1.0 / (1.0 + jnp.exp(-s)), EPS, 1.0 - EPS)
    yv = y_ref[...]
    bce = -(yv * jnp.log(yp) + (1.0 - yv) * jnp.log(1.0 - yp))

    dp = hp3 + re_p - tp3
    dn = hn3 + re_n - tn3
    ypos = jnp.sqrt(jnp.sum(dp * dp, axis=2))
    yneg = jnp.sqrt(jnp.sum(dn * dn, axis=2))
    hinge = jnp.maximum(ypos - yneg + MARGIN, 0.0)

    batchf = jnp.float32(CBR * 128 * nsteps)

    @pl.when(k == 0)
    def _():
        o_ref[0, 0] = 0.0

    o_ref[0, 0] += jnp.sum(bce) / batchf + ALPHA * jnp.sum(hinge)


def kernel(u, i, y, h_pos, r_pos, t_pos, h_neg, r_neg, t_neg, u_emb, e_emb, r_emb):
    batch = u.shape[0]
    et = e_emb.T
    ut = u_emb.T
    ut_tail = u_emb[99968:, :].T
    et_tail = e_emb[999936:, :].T

    gather_call = _make_gather_call(batch)
    rows = gather_call(u.astype(jnp.int32), i.astype(jnp.int32),
                       h_pos.astype(jnp.int32), t_pos.astype(jnp.int32),
                       h_neg.astype(jnp.int32), t_neg.astype(jnp.int32),
                       ut, et, ut_tail, et_tail)
    rows3 = [r.reshape((batch + 128) // 128, 128, 2 * E) for r in rows]

    nsteps = batch // (CBR * 128)
    rows_spec = pl.BlockSpec((CBR, 128, 2 * E), lambda k: (k, 0, 0))
    vec2d = (batch // 128, 128)
    vec_spec = pl.BlockSpec((CBR, 128), lambda k: (k, 0))
    tab_spec = pl.BlockSpec((E, E), lambda k: (0, 0))

    out = pl.pallas_call(
        _tc_body,
        grid=(nsteps,),
        in_specs=[vec_spec, vec_spec, vec_spec] + [rows_spec] * 6 + [tab_spec],
        out_specs=pl.BlockSpec((1, 1), lambda k: (0, 0),
                               memory_space=pltpu.SMEM),
        out_shape=jax.ShapeDtypeStruct((1, 1), jnp.float32),
    )(r_pos.astype(jnp.int32).reshape(vec2d),
      r_neg.astype(jnp.int32).reshape(vec2d),
      y.reshape(vec2d),
      *rows3, r_emb)
    return out[0, 0]


# speedup vs baseline: 5.1290x; 5.1290x over previous
import functools

import jax
import jax.numpy as jnp
from jax import lax
from jax.experimental import pallas as pl
from jax.experimental.pallas import tpu as pltpu
from jax.experimental.pallas import tpu_sc as plsc

E = 64
L = 16
NC = 2
NS = 16
NW = NC * NS
MARGIN = 1.0
ALPHA = 0.2
EPS = 1e-7


def _sc_body(u, i, hp, rp, tp, hn, rn, tn, u_emb, e_emb, r_emb,
             s_out, sqp_out, sqn_out,
             idx_u, idx_i, idx_hp, idx_tp, idx_hn, idx_tn, idx_rp, idx_rn,
             r_tab,
             rows_u0, rows_i0, rows_hp0, rows_tp0, rows_hn0, rows_tn0,
             rows_u1, rows_i1, rows_hp1, rows_tp1, rows_hn1, rows_tn1,
             svec, pvec, nvec, sem0, sem1, isem):
    bpw = svec.shape[0]
    c_rows = rows_u0.shape[0]
    n_chunks = bpw // c_rows
    groups = c_rows // L

    wid = lax.axis_index("s") * NC + lax.axis_index("c")
    base = wid * bpw
    sl_w = pl.ds(base, bpw)

    stage = [
        pltpu.async_copy(u.at[sl_w], idx_u, isem),
        pltpu.async_copy(i.at[sl_w], idx_i, isem),
        pltpu.async_copy(hp.at[sl_w], idx_hp, isem),
        pltpu.async_copy(tp.at[sl_w], idx_tp, isem),
        pltpu.async_copy(hn.at[sl_w], idx_hn, isem),
        pltpu.async_copy(tn.at[sl_w], idx_tn, isem),
        pltpu.async_copy(rp.at[sl_w], idx_rp, isem),
        pltpu.async_copy(rn.at[sl_w], idx_rn, isem),
        pltpu.async_copy(r_emb, r_tab, isem),
    ]
    for cp in stage:
        cp.wait()

    bufs = (
        (rows_u0, rows_i0, rows_hp0, rows_tp0, rows_hn0, rows_tn0, sem0),
        (rows_u1, rows_i1, rows_hp1, rows_tp1, rows_hn1, rows_tn1, sem1),
    )

    def issue(c):
        ru, ri, rhp, rtp, rhn, rtn, sem = bufs[c % 2]
        sl = pl.ds(c * c_rows, c_rows)
        return [
            pltpu.async_copy(u_emb.at[idx_u.at[sl]], ru, sem),
            pltpu.async_copy(e_emb.at[idx_i.at[sl]], ri, sem),
            pltpu.async_copy(e_emb.at[idx_hp.at[sl]], rhp, sem),
            pltpu.async_copy(e_emb.at[idx_tp.at[sl]], rtp, sem),
            pltpu.async_copy(e_emb.at[idx_hn.at[sl]], rhn, sem),
            pltpu.async_copy(e_emb.at[idx_tn.at[sl]], rtn, sem),
        ]

    iota = lax.iota(jnp.int32, L)
    pend = {0: issue(0)}
    for c in range(n_chunks):
        if c + 1 < n_chunks:
            pend[c + 1] = issue(c + 1)
        for cp in pend.pop(c):
            cp.wait()
        ru, ri, rhp, rtp, rhn, rtn, _ = bufs[c % 2]
        cbase = c * c_rows

        def group_body(g, _, ru=ru, ri=ri, rhp=rhp, rtp=rtp, rhn=rhn,
                       rtn=rtn, cbase=cbase):
            goff = pl.multiple_of(g * L, L)
            row = goff + iota
            sl16 = pl.ds(cbase + goff, L)
            rp_v = idx_rp[sl16]
            rn_v = idx_rn[sl16]

            def d_body(dd, accs):
                acc_s, acc_p, acc_n = accs
                sp, pp, np_ = [], [], []
                for k in range(8):
                    d = dd * 8 + k
                    col = jnp.full((L,), d, jnp.int32)
                    ue = plsc.load_gather(ru, [row, col])
                    ie = plsc.load_gather(ri, [row, col])
                    sp.append(ue * ie)
                    hpe = plsc.load_gather(rhp, [row, col])
                    tpe = plsc.load_gather(rtp, [row, col])
                    rpe = plsc.load_gather(r_tab, [rp_v, col])
                    dp = hpe + rpe - tpe
                    pp.append(dp * dp)
                    hne = plsc.load_gather(rhn, [row, col])
                    tne = plsc.load_gather(rtn, [row, col])
                    rne = plsc.load_gather(r_tab, [rn_v, col])
                    dn = hne + rne - tne
                    np_.append(dn * dn)

                def tree(parts):
                    while len(parts) > 1:
                        parts = [a + b for a, b in
                                 zip(parts[::2], parts[1::2])]
                    return parts[0]

                return (acc_s + tree(sp), acc_p + tree(pp),
                        acc_n + tree(np_))

            zero = jnp.zeros((L,), jnp.float32)
            acc_s, acc_p, acc_n = lax.fori_loop(0, E // 8, d_body,
                                                (zero, zero, zero))
            svec[sl16] = acc_s
            pvec[sl16] = acc_p
            nvec[sl16] = acc_n
            return 0

        lax.fori_loop(0, groups, group_body, 0)

    pltpu.sync_copy(svec, s_out.at[sl_w])
    pltpu.sync_copy(pvec, sqp_out.at[sl_w])
    pltpu.sync_copy(nvec, sqn_out.at[sl_w])


def _make_sc_call(batch):
    bpw = batch // NW
    c_rows = min(bpw, 64)
    mesh = plsc.VectorSubcoreMesh(core_axis_name="c", subcore_axis_name="s")
    f32 = jnp.float32
    return pl.kernel(
        _sc_body,
        out_type=[jax.ShapeDtypeStruct((batch,), f32)] * 3,
        mesh=mesh,
        compiler_params=pltpu.CompilerParams(
            needs_layout_passes=False, use_tc_tiling_on_sc=True),
        scratch_types=(
            [pltpu.VMEM((bpw,), jnp.int32)] * 8
            + [pltpu.VMEM((64, 2 * E), f32)]
            + [pltpu.VMEM((c_rows, 2 * E), f32)] * 12
            + [pltpu.VMEM((bpw,), f32)] * 3
            + [pltpu.SemaphoreType.DMA] * 3
        ),
    )


def _finish_body(y_ref, s_ref, p_ref, n_ref, o_ref):
    s = s_ref[...]
    yp = jnp.clip(1.0 / (1.0 + jnp.exp(-s)), EPS, 1.0 - EPS)
    yv = y_ref[...]
    bce = -(yv * jnp.log(yp) + (1.0 - yv) * jnp.log(1.0 - yp))
    ypos = jnp.sqrt(p_ref[...])
    yneg = jnp.sqrt(n_ref[...])
    hinge = jnp.maximum(ypos - yneg + MARGIN, 0.0)
    n = s.shape[0] * s.shape[1]
    o_ref[0, 0] = jnp.sum(bce) / n + ALPHA * jnp.sum(hinge)


def kernel(u, i, y, h_pos, r_pos, t_pos, h_neg, r_neg, t_neg, u_emb, e_emb, r_emb):
    batch = u.shape[0]
    e_dim = u_emb.shape[1]
    pad = [(0, 0), (0, e_dim)]
    u2 = jnp.pad(u_emb, pad)
    e2 = jnp.pad(e_emb, pad)
    r2 = jnp.pad(r_emb, pad)
    sc_call = _make_sc_call(batch)
    s, sqp, sqn = sc_call(u.astype(jnp.int32), i.astype(jnp.int32),
                          h_pos.astype(jnp.int32), r_pos.astype(jnp.int32),
                          t_pos.astype(jnp.int32), h_neg.astype(jnp.int32),
                          r_neg.astype(jnp.int32), t_neg.astype(jnp.int32),
                          u2, e2, r2)
    rows = batch // 128
    shape2d = (rows, 128)
    out = pl.pallas_call(
        _finish_body,
        out_shape=jax.ShapeDtypeStruct((1, 1), jnp.float32),
        out_specs=pl.BlockSpec(memory_space=pltpu.SMEM),
    )(y.reshape(shape2d), s.reshape(shape2d), sqp.reshape(shape2d),
      sqn.reshape(shape2d))
    return out[0, 0]
